# Initial kernel scaffold; baseline (speedup 1.0000x reference)
import jax
import jax.numpy as jnp
from jax.experimental import pallas as pl


def kernel(ids, tables):
    raise NotImplementedError("write your pallas kernel here")


# speedup vs baseline: 3.5051x; 3.5051x over previous
import functools

import jax
import jax.numpy as jnp
from jax import lax
from jax.experimental import pallas as pl
from jax.experimental.pallas import tpu as pltpu
from jax.experimental.pallas import tpu_sc as plsc

_F = 26
_V = 100000
_D = 32
_B = 1024
_S = 50

_N = _F * _B * _S
_NW = 32
_PER_W = _N // _NW
_K = 13
_RB = _K * 128
_NBLK = _PER_W // _RB


def _gather_body(table_hbm, gidx_hbm, out_hbm, idx_v, rows_v, sem):
    wid = lax.axis_index("s") * 2 + lax.axis_index("c")

    def block(i, carry):
        pltpu.sync_copy(gidx_hbm.at[wid, i], idx_v)
        copies = [
            pltpu.async_copy(
                table_hbm.at[idx_v.at[j]],
                rows_v.at[pl.ds(j * 128, 128)],
                sem,
            )
            for j in range(_K)
        ]
        for c in copies:
            c.wait()
        pltpu.sync_copy(rows_v, out_hbm.at[wid, i])
        return carry

    lax.fori_loop(0, _NBLK, block, 0)


def kernel(ids, tables):
    offs = (jnp.arange(_F, dtype=jnp.int32) * _V)[:, None, None]
    gidx = (ids.astype(jnp.int32) + offs).transpose(1, 2, 0)
    gidx = gidx.reshape(_NW, _NBLK, _K, 128)
    table = tables.reshape(_F * _V, _D)

    mesh = plsc.VectorSubcoreMesh(core_axis_name="c", subcore_axis_name="s")
    out = pl.kernel(
        _gather_body,
        out_type=jax.ShapeDtypeStruct((_NW, _NBLK, _RB, _D), jnp.float32),
        mesh=mesh,
        scratch_types=[
            pltpu.VMEM((_K, 128), jnp.int32),
            pltpu.VMEM((_RB, _D), jnp.float32),
            pltpu.SemaphoreType.DMA,
        ],
        compiler_params=pltpu.CompilerParams(use_tc_tiling_on_sc=False),
    )(table, gidx)

    out = out.reshape(_B, _S, _F * _D)
    seq_len = jnp.full((_B,), _S, dtype=ids.dtype)
    return out, seq_len
